# baseline (speedup 1.0000x reference)
import jax
import jax.numpy as jnp
from jax import lax
from jax.experimental import pallas as pl
from jax.experimental.pallas import tpu as pltpu
from jax.experimental.pallas import tpu_sc as plsc

_N = 2048
_K = 16
_RB = 512
_PAD = 16
_NW = 32
_CH = _N * _K // _NW


def _topk_body(xt_ref, c_ref, idx_ref):
    xt = xt_ref[...]
    c = c_ref[...]
    xr = xt[0:1, :]
    yr = xt[1:2, :]
    zr = xt[2:3, :]
    cx = c[:, 0:1]
    cy = c[:, 1:2]
    cz = c[:, 2:3]
    d = (cx - xr) ** 2 + (cy - yr) ** 2 + (cz - zr) ** 2
    q = _N // 2
    iotaf = lax.broadcasted_iota(jnp.int32, (_RB, q), 1).astype(jnp.float32)
    s0, s1 = d[:, :q], d[:, q:]
    i0, i1 = iotaf, iotaf + jnp.float32(q)
    swap = s1 < s0
    s0, s1 = jnp.where(swap, s1, s0), jnp.where(swap, s0, s1)
    i0, i1 = jnp.where(swap, i1, i0), jnp.where(swap, i0, i1)

    jlane = lax.broadcasted_iota(jnp.int32, (_RB, _K), 1)
    big_c = jnp.float32(2 * _N)
    inf = jnp.float32(jnp.inf)
    out = jnp.zeros((_RB, _K), jnp.float32)
    for j in range(_K):
        m = jnp.min(s0, axis=1, keepdims=True)
        cand = jnp.where(s0 == m, i0, big_c)
        amin = jnp.min(cand, axis=1, keepdims=True)
        win = i0 == amin
        out = jnp.where(jlane == j, amin, out)
        s0 = jnp.where(win, s1, s0)
        i0 = jnp.where(win, i1, i0)
        s1 = jnp.where(win, inf, s1)
    idx_ref[...] = out.astype(jnp.int32)


def _sc_gather(table_hbm, idx_hbm, out_hbm, idx_v, rows_v, cent_v, sem):
    wid = lax.axis_index("s") * 2 + lax.axis_index("c")
    base = pl.multiple_of(wid * _CH, _CH)
    pltpu.sync_copy(
        idx_hbm.at[pl.ds(pl.multiple_of(base // 128, _CH // 128), _CH // 128)],
        idx_v)
    copies = []
    for j in range(_CH // 128):
        copies.append(pltpu.async_copy(
            table_hbm.at[idx_v.at[j]],
            rows_v.at[pl.ds(j * 128, 128)], sem))
    pltpu.sync_copy(
        table_hbm.at[pl.ds(pl.multiple_of(base // _K, _CH // _K), _CH // _K)],
        cent_v)
    for cp in copies:
        cp.wait()

    def body(g, _):
        cv = cent_v[g]
        for mm in range(_K):
            r = g * _K + mm
            rows_v[r] = rows_v[r] - cv
        return 0

    lax.fori_loop(0, _CH // _K, body, 0)
    pltpu.sync_copy(rows_v, out_hbm.at[pl.ds(base, _CH)])


def kernel(pcd):
    b, n, _ = pcd.shape
    mesh = plsc.VectorSubcoreMesh(core_axis_name="c", subcore_axis_name="s")

    topk = pl.pallas_call(
        _topk_body,
        grid=(n // _RB,),
        in_specs=[
            pl.BlockSpec((3, _N), lambda r: (0, 0)),
            pl.BlockSpec((_RB, 3), lambda r: (r, 0)),
        ],
        out_specs=pl.BlockSpec((_RB, _K), lambda r: (r, 0)),
        out_shape=jax.ShapeDtypeStruct((n, _K), jnp.int32),
    )

    gather = pl.kernel(
        _sc_gather,
        mesh=mesh,
        out_type=jax.ShapeDtypeStruct((n * _K, _PAD), jnp.float32),
        scratch_types=[
            pltpu.VMEM((_CH // 128, 128), jnp.int32),
            pltpu.VMEM((_CH, _PAD), jnp.float32),
            pltpu.VMEM((_CH // _K, _PAD), jnp.float32),
            pltpu.SemaphoreType.DMA,
        ],
        compiler_params=pltpu.CompilerParams(use_tc_tiling_on_sc=False),
    )

    nbs = []
    for i in range(b):
        xyz = pcd[i]
        xt = jnp.transpose(xyz, (1, 0))
        idx = topk(xt, xyz)
        table = jnp.pad(xyz, ((0, 0), (0, _PAD - 3)))
        nb = gather(table, idx.reshape(n * _K // 128, 128))
        nbs.append(nb.reshape(n, _K, _PAD)[..., :3])

    neighborhood = jnp.stack(nbs, axis=0)
    return neighborhood, pcd
